# baseline (speedup 1.0000x reference)
import jax
import jax.numpy as jnp
from jax.experimental import pallas as pl

_K = 1024
_D = 64
_BN = 256
_KC = 128


def _quantize_one(x, e_ref):
    lane = jax.lax.broadcasted_iota(jnp.int32, (_BN, _KC), 1)
    x_sq = jnp.sum(x * x, axis=1, keepdims=True)

    def chunk(kc, carry):
        minval, minidx = carry
        emb_c = e_ref[pl.ds(kc * _KC, _KC), :]
        mm = jax.lax.dot_general(
            x, emb_c, (((1,), (1,)), ((), ())),
            preferred_element_type=jnp.float32)
        emb_sq = jnp.sum(emb_c * emb_c, axis=1)
        dist = (emb_sq[None, :] - 2.0 * mm) + x_sq
        cmin = jnp.min(dist, axis=1, keepdims=True)
        cidx = jnp.min(jnp.where(dist == cmin, lane, _KC),
                       axis=1, keepdims=True) + kc * _KC
        upd = cmin < minval
        return (jnp.where(upd, cmin, minval),
                jnp.where(upd, cidx, minidx))

    init = (jnp.full((_BN, 1), jnp.inf, jnp.float32),
            jnp.zeros((_BN, 1), jnp.int32))
    _, idx = jax.lax.fori_loop(0, _K // _KC, chunk, init)

    kiota = jax.lax.broadcasted_iota(jnp.int32, (_BN, _K), 1)
    onehot = (kiota == idx).astype(jnp.float32)
    return jax.lax.dot_general(
        onehot, e_ref[...], (((1,), (0,)), ((), ())),
        preferred_element_type=jnp.float32)


def _body(x1_ref, x0_ref, e1_ref, e0_ref, o_ref):
    o_ref[:, :_D] = _quantize_one(x1_ref[...], e1_ref)
    o_ref[:, _D:] = _quantize_one(x0_ref[...], e0_ref)


def kernel(enc0, enc1, codebook0, codebook1):
    B, T, d = enc0.shape
    n = B * T
    flat1 = enc1.reshape(n, d)
    flat0 = enc0.reshape(n, d)
    out = pl.pallas_call(
        _body,
        grid=(n // _BN,),
        in_specs=[
            pl.BlockSpec((_BN, _D), lambda i: (i, 0)),
            pl.BlockSpec((_BN, _D), lambda i: (i, 0)),
            pl.BlockSpec((_K, _D), lambda i: (0, 0)),
            pl.BlockSpec((_K, _D), lambda i: (0, 0)),
        ],
        out_specs=pl.BlockSpec((_BN, 2 * _D), lambda i: (i, 0)),
        out_shape=jax.ShapeDtypeStruct((n, 2 * _D), jnp.float32),
    )(flat1, flat0, codebook1, codebook0)
    return out.reshape(B, T, 2 * d)


# speedup vs baseline: 99.1493x; 99.1493x over previous
import jax
import jax.numpy as jnp
from jax.experimental import pallas as pl

_K = 1024
_D = 64
_BN = 256


def _quantize_one(x, e_ref):
    e = e_ref[...]
    x_sq = jnp.sum(x * x, axis=1, keepdims=True)
    ones = jnp.ones((1, _D), jnp.float32)
    e_sq = jax.lax.dot_general(
        ones, e * e, (((1,), (1,)), ((), ())),
        preferred_element_type=jnp.float32)
    mm = jax.lax.dot_general(
        x, e, (((1,), (1,)), ((), ())),
        preferred_element_type=jnp.float32)
    dist = (e_sq - 2.0 * mm) + x_sq
    gmin = jnp.min(dist, axis=1, keepdims=True)
    kiota = jax.lax.broadcasted_iota(jnp.int32, (x.shape[0], _K), 1)
    idx = jnp.min(jnp.where(dist == gmin, kiota, _K),
                  axis=1, keepdims=True)
    onehot = (kiota == idx).astype(jnp.float32)
    return jax.lax.dot_general(
        onehot, e, (((1,), (0,)), ((), ())),
        preferred_element_type=jnp.float32)


def _body(x1_ref, x0_ref, e1_ref, e0_ref, o_ref):
    o_ref[:, :_D] = _quantize_one(x1_ref[...], e1_ref)
    o_ref[:, _D:] = _quantize_one(x0_ref[...], e0_ref)


def kernel(enc0, enc1, codebook0, codebook1):
    B, T, d = enc0.shape
    n = B * T
    flat1 = enc1.reshape(n, d)
    flat0 = enc0.reshape(n, d)
    out = pl.pallas_call(
        _body,
        grid=(n // _BN,),
        in_specs=[
            pl.BlockSpec((_BN, _D), lambda i: (i, 0)),
            pl.BlockSpec((_BN, _D), lambda i: (i, 0)),
            pl.BlockSpec((_K, _D), lambda i: (0, 0)),
            pl.BlockSpec((_K, _D), lambda i: (0, 0)),
        ],
        out_specs=pl.BlockSpec((_BN, 2 * _D), lambda i: (i, 0)),
        out_shape=jax.ShapeDtypeStruct((n, 2 * _D), jnp.float32),
    )(flat1, flat0, codebook1, codebook0)
    return out.reshape(B, T, 2 * d)


# speedup vs baseline: 124.6559x; 1.2573x over previous
import jax
import jax.numpy as jnp
from jax.experimental import pallas as pl

_K = 1024
_D = 64
_BN = 512


def _quantize_one(x, e_ref):
    e = e_ref[...]
    x_sq = jnp.sum(x * x, axis=1, keepdims=True)
    ones = jnp.ones((1, _D), jnp.float32)
    e_sq = jax.lax.dot_general(
        ones, e * e, (((1,), (1,)), ((), ())),
        preferred_element_type=jnp.float32)
    mm = jax.lax.dot_general(
        x, e, (((1,), (1,)), ((), ())),
        preferred_element_type=jnp.float32)
    dist = (e_sq - 2.0 * mm) + x_sq
    gmin = jnp.min(dist, axis=1, keepdims=True)
    kiota = jax.lax.broadcasted_iota(jnp.int32, (x.shape[0], _K), 1)
    idx = jnp.min(jnp.where(dist == gmin, kiota, _K),
                  axis=1, keepdims=True)
    onehot = (kiota == idx).astype(jnp.float32)
    return jax.lax.dot_general(
        onehot, e, (((1,), (0,)), ((), ())),
        preferred_element_type=jnp.float32)


def _body(x1_ref, x0_ref, e1_ref, e0_ref, o_ref):
    o_ref[:, :_D] = _quantize_one(x1_ref[...], e1_ref)
    o_ref[:, _D:] = _quantize_one(x0_ref[...], e0_ref)


def kernel(enc0, enc1, codebook0, codebook1):
    B, T, d = enc0.shape
    n = B * T
    flat1 = enc1.reshape(n, d)
    flat0 = enc0.reshape(n, d)
    out = pl.pallas_call(
        _body,
        grid=(n // _BN,),
        in_specs=[
            pl.BlockSpec((_BN, _D), lambda i: (i, 0)),
            pl.BlockSpec((_BN, _D), lambda i: (i, 0)),
            pl.BlockSpec((_K, _D), lambda i: (0, 0)),
            pl.BlockSpec((_K, _D), lambda i: (0, 0)),
        ],
        out_specs=pl.BlockSpec((_BN, 2 * _D), lambda i: (i, 0)),
        out_shape=jax.ShapeDtypeStruct((n, 2 * _D), jnp.float32),
    )(flat1, flat0, codebook1, codebook0)
    return out.reshape(B, T, 2 * d)
